# baseline (speedup 1.0000x reference)
import functools

import jax
import jax.numpy as jnp
from jax import lax
from jax.experimental import pallas as pl
from jax.experimental.pallas import tpu as pltpu
from jax.experimental.pallas import tpu_sc as plsc

_V, _D, _H = 100000, 64, 128
_B, _L = 4096, 50
_NC, _NS = 2, 16
_NW = _NC * _NS
_RPW = _B // _NW
_CPW = _RPW // 2
_CI = 2 * _L
_CIP = 104
_F = 3 * _D
_OPW = _RPW * _F


def _sc_body(x3, e1, e2, e3, out, idxs, b0a, b0b, b0c, b1a, b1b, b1c,
             acc_out, sem0, sem1):
    wid = lax.axis_index("s") * _NC + lax.axis_index("c")
    pltpu.sync_copy(x3.at[wid], idxs)

    tables = (e1, e2, e3)
    bufs0 = (b0a, b0b, b0c)
    bufs1 = (b1a, b1b, b1c)

    def fire(c, bufs, sem):
        for t in range(3):
            pltpu.async_copy(tables[t].at[idxs.at[c, pl.ds(0, _CIP)]],
                             bufs[t], sem)

    def drain(c, bufs, sem):
        for t in range(3):
            pltpu.make_async_copy(tables[t].at[idxs.at[c, pl.ds(0, _CIP)]],
                                  bufs[t], sem).wait()

    def reduce_chunk(c, bufs):
        for br in range(2):
            base = br * _L
            orow = 2 * c + br
            for t in range(3):
                buf = bufs[t]

                def body(r, accs, buf=buf, base=base):
                    rr = base + r * 5
                    a0, a1, a2, a3 = accs
                    for u in range(5):
                        a0 = a0 + buf[rr + u, pl.ds(0, 16)]
                        a1 = a1 + buf[rr + u, pl.ds(16, 16)]
                        a2 = a2 + buf[rr + u, pl.ds(32, 16)]
                        a3 = a3 + buf[rr + u, pl.ds(48, 16)]
                    return (a0, a1, a2, a3)

                z = jnp.zeros((16,), jnp.float32)
                accs = lax.fori_loop(0, _L // 5, body, (z, z, z, z))
                for j in range(4):
                    acc_out[pl.ds(orow * _F + t * _D + j * 16, 16)] = accs[j]

    fire(0, bufs0, sem0)

    def body2(i, carry):
        c0 = 2 * i
        fire(c0 + 1, bufs1, sem1)
        drain(c0, bufs0, sem0)
        reduce_chunk(c0, bufs0)

        @pl.when(i < _CPW // 2 - 1)
        def _():
            fire(c0 + 2, bufs0, sem0)

        drain(c0 + 1, bufs1, sem1)
        reduce_chunk(c0 + 1, bufs1)
        return carry

    lax.fori_loop(0, _CPW // 2, body2, 0)
    pltpu.sync_copy(acc_out, out.at[wid])


_sc_pool = functools.partial(
    pl.kernel,
    mesh=plsc.VectorSubcoreMesh(core_axis_name="c", subcore_axis_name="s"),
    out_type=jax.ShapeDtypeStruct((_NW, _OPW), jnp.float32),
    scratch_types=[
        pltpu.VMEM((_CPW, 128), jnp.int32),
        pltpu.VMEM((_CIP, _D), jnp.float32),
        pltpu.VMEM((_CIP, _D), jnp.float32),
        pltpu.VMEM((_CIP, _D), jnp.float32),
        pltpu.VMEM((_CIP, _D), jnp.float32),
        pltpu.VMEM((_CIP, _D), jnp.float32),
        pltpu.VMEM((_CIP, _D), jnp.float32),
        pltpu.VMEM((_OPW,), jnp.float32),
        pltpu.SemaphoreType.DMA,
        pltpu.SemaphoreType.DMA,
    ],
    compiler_params=pltpu.CompilerParams(use_tc_tiling_on_sc=False),
)(_sc_body)


def _tc_body(pooled_ref, x_ref, e1_ref, fc1w_ref, fc1b_ref, gamma_ref,
             beta_ref, fc2w_ref, fc2b_ref, out_ref):
    x = x_ref[...]
    cnt0 = jnp.sum((x == 0).astype(jnp.float32), axis=1,
                   keepdims=True)
    row0 = jnp.concatenate(
        [e1_ref[0:1, :], jnp.zeros((1, 2 * _D), jnp.float32)],
        axis=1)
    feat = (pooled_ref[...] - cnt0 * row0) * (1.0 / _L)
    z = lax.dot_general(feat, fc1w_ref[...], (((1,), (1,)), ((), ())),
                        preferred_element_type=jnp.float32)
    z = jnp.maximum(z + fc1b_ref[...], 0.0)
    m = jnp.mean(z, axis=0, keepdims=True)
    v = jnp.mean((z - m) * (z - m), axis=0, keepdims=True)
    zn = (z - m) * lax.rsqrt(v + 1e-5) * gamma_ref[...] + beta_ref[...]
    out_ref[...] = lax.dot_general(
        zn, fc2w_ref[...], (((1,), (1,)), ((), ())),
        preferred_element_type=jnp.float32) + fc2b_ref[...]


_tc_mlp = pl.pallas_call(
    _tc_body,
    grid=(1,),
    in_specs=[
        pl.BlockSpec((_B, _F), lambda i: (0, 0)),
        pl.BlockSpec((_B, _L), lambda i: (0, 0)),
        pl.BlockSpec((8, _D), lambda i: (0, 0)),
        pl.BlockSpec((_H, _F), lambda i: (0, 0)),
        pl.BlockSpec((1, _H), lambda i: (0, 0)),
        pl.BlockSpec((1, _H), lambda i: (0, 0)),
        pl.BlockSpec((1, _H), lambda i: (0, 0)),
        pl.BlockSpec((2, _H), lambda i: (0, 0)),
        pl.BlockSpec((1, 2), lambda i: (0, 0)),
    ],
    out_specs=pl.BlockSpec((_B, 2), lambda i: (0, 0)),
    out_shape=jax.ShapeDtypeStruct((_B, 2), jnp.float32),
)


def kernel(x, s, emb1, emb2, emb3, fc1_w, fc1_b, gamma, beta, fc2_w, fc2_b):
    x = x.astype(jnp.int32)
    x3 = x.reshape(_NW, _CPW, _CI)
    xp = jnp.concatenate(
        [x3, jnp.zeros((_NW, _CPW, 128 - _CI), jnp.int32)], axis=2)
    pooled = _sc_pool(xp, emb1, emb2, emb3).reshape(_B, _F)
    e1row = lax.slice(emb1, (0, 0), (8, _D))
    return _tc_mlp(pooled, x, e1row, fc1_w, fc1_b.reshape(1, _H),
                   gamma.reshape(1, _H), beta.reshape(1, _H), fc2_w,
                   fc2_b.reshape(1, 2))


# speedup vs baseline: 1.4985x; 1.4985x over previous
import functools

import jax
import jax.numpy as jnp
from jax import lax
from jax.experimental import pallas as pl
from jax.experimental.pallas import tpu as pltpu
from jax.experimental.pallas import tpu_sc as plsc

_V, _D, _H = 100000, 64, 128
_B, _L = 4096, 50
_NC, _NS = 2, 16
_NW = _NC * _NS
_RPW = _B // _NW
_CPW = _RPW // 2
_CI = 2 * _L


def _sc_body(x3, e1, e2, e3, out, idxs, b0a, b0b, b0c, b1a, b1b, b1c,
             acc_out, sem0, sem1):
    wid = lax.axis_index("s") * _NC + lax.axis_index("c")
    pltpu.sync_copy(x3.at[wid], idxs)

    tables = (e1, e2, e3)
    bufs0 = (b0a, b0b, b0c)
    bufs1 = (b1a, b1b, b1c)

    def fire(c, bufs, sem):
        for t in range(3):
            pltpu.async_copy(tables[t].at[idxs.at[c]], bufs[t], sem)

    def drain(c, bufs, sem):
        for t in range(3):
            pltpu.make_async_copy(tables[t].at[idxs.at[c]], bufs[t],
                                  sem).wait()

    def reduce_chunk(c, bufs):
        for br in range(2):
            base = br * _L
            orow = 2 * c + br
            for t in range(3):
                buf = bufs[t]

                def body(r, accs, buf=buf, base=base):
                    rr = base + r * 5
                    a0, a1, a2, a3 = accs
                    for u in range(5):
                        a0 = a0 + buf[rr + u, pl.ds(0, 16)]
                        a1 = a1 + buf[rr + u, pl.ds(16, 16)]
                        a2 = a2 + buf[rr + u, pl.ds(32, 16)]
                        a3 = a3 + buf[rr + u, pl.ds(48, 16)]
                    return (a0, a1, a2, a3)

                z = jnp.zeros((16,), jnp.float32)
                accs = lax.fori_loop(0, _L // 5, body, (z, z, z, z))
                for j in range(4):
                    acc_out[orow, pl.ds(t * _D + j * 16, 16)] = accs[j]

    fire(0, bufs0, sem0)

    def body2(i, carry):
        c0 = 2 * i
        fire(c0 + 1, bufs1, sem1)
        drain(c0, bufs0, sem0)
        reduce_chunk(c0, bufs0)

        @pl.when(i < _CPW // 2 - 1)
        def _():
            fire(c0 + 2, bufs0, sem0)

        drain(c0 + 1, bufs1, sem1)
        reduce_chunk(c0 + 1, bufs1)
        return carry

    lax.fori_loop(0, _CPW // 2, body2, 0)
    pltpu.sync_copy(acc_out, out.at[wid])


_sc_pool = functools.partial(
    pl.kernel,
    mesh=plsc.VectorSubcoreMesh(core_axis_name="c", subcore_axis_name="s"),
    out_type=jax.ShapeDtypeStruct((_NW, _RPW, 3 * _D), jnp.float32),
    scratch_types=[
        pltpu.VMEM((_CPW, _CI), jnp.int32),
        pltpu.VMEM((_CI, _D), jnp.float32),
        pltpu.VMEM((_CI, _D), jnp.float32),
        pltpu.VMEM((_CI, _D), jnp.float32),
        pltpu.VMEM((_CI, _D), jnp.float32),
        pltpu.VMEM((_CI, _D), jnp.float32),
        pltpu.VMEM((_CI, _D), jnp.float32),
        pltpu.VMEM((_RPW, 3 * _D), jnp.float32),
        pltpu.SemaphoreType.DMA,
        pltpu.SemaphoreType.DMA,
    ],
    compiler_params=pltpu.CompilerParams(use_tc_tiling_on_sc=False),
)(_sc_body)


def _tc_body(pooled_ref, x_ref, e1_ref, fc1w_ref, fc1b_ref, gamma_ref,
             beta_ref, fc2w_ref, fc2b_ref, out_ref):
    x = x_ref[...]
    cnt0 = jnp.sum((x == 0).astype(jnp.float32), axis=1,
                   keepdims=True)
    row0 = jnp.concatenate(
        [e1_ref[0:1, :], jnp.zeros((1, 2 * _D), jnp.float32)],
        axis=1)
    feat = (pooled_ref[...] - cnt0 * row0) * (1.0 / _L)
    z = lax.dot_general(feat, fc1w_ref[...], (((1,), (1,)), ((), ())),
                        preferred_element_type=jnp.float32)
    z = jnp.maximum(z + fc1b_ref[...], 0.0)
    m = jnp.mean(z, axis=0, keepdims=True)
    v = jnp.mean((z - m) * (z - m), axis=0, keepdims=True)
    zn = (z - m) * lax.rsqrt(v + 1e-5) * gamma_ref[...] + beta_ref[...]
    out_ref[...] = lax.dot_general(
        zn, fc2w_ref[...], (((1,), (1,)), ((), ())),
        preferred_element_type=jnp.float32) + fc2b_ref[...]


_tc_mlp = pl.pallas_call(
    _tc_body,
    grid=(1,),
    in_specs=[
        pl.BlockSpec((_B, 3 * _D), lambda i: (0, 0)),
        pl.BlockSpec((_B, _L), lambda i: (0, 0)),
        pl.BlockSpec((8, _D), lambda i: (0, 0)),
        pl.BlockSpec((_H, 3 * _D), lambda i: (0, 0)),
        pl.BlockSpec((1, _H), lambda i: (0, 0)),
        pl.BlockSpec((1, _H), lambda i: (0, 0)),
        pl.BlockSpec((1, _H), lambda i: (0, 0)),
        pl.BlockSpec((2, _H), lambda i: (0, 0)),
        pl.BlockSpec((1, 2), lambda i: (0, 0)),
    ],
    out_specs=pl.BlockSpec((_B, 2), lambda i: (0, 0)),
    out_shape=jax.ShapeDtypeStruct((_B, 2), jnp.float32),
)


def kernel(x, s, emb1, emb2, emb3, fc1_w, fc1_b, gamma, beta, fc2_w, fc2_b):
    x = x.astype(jnp.int32)
    x3 = x.reshape(_NW, _CPW, _CI)
    pooled = _sc_pool(x3, emb1, emb2, emb3).reshape(_B, 3 * _D)
    return _tc_mlp(pooled, x, emb1, fc1_w, fc1_b.reshape(1, _H),
                   gamma.reshape(1, _H), beta.reshape(1, _H), fc2_w,
                   fc2_b.reshape(1, 2))


# speedup vs baseline: 1.6552x; 1.1046x over previous
import functools

import jax
import jax.numpy as jnp
from jax import lax
from jax.experimental import pallas as pl
from jax.experimental.pallas import tpu as pltpu
from jax.experimental.pallas import tpu_sc as plsc

_V, _D, _H = 100000, 64, 128
_B, _L = 4096, 50
_NC, _NS = 2, 16
_NW = _NC * _NS
_RPW = _B // _NW
_CPW = _RPW // 2
_CI = 2 * _L


def _pool_one_table(x3, emb, out, idxs, b0, b1, acc, sem0, sem1):
    wid = lax.axis_index("s") * _NC + lax.axis_index("c")
    pltpu.sync_copy(x3.at[wid], idxs)

    def fire(c, buf, sem):
        pltpu.async_copy(emb.at[idxs.at[c]], buf, sem)

    def drain(c, buf, sem):
        pltpu.make_async_copy(emb.at[idxs.at[c]], buf, sem).wait()

    def reduce_chunk(c, buf):
        for br in range(2):
            base = br * _L
            orow = 2 * c + br

            def body(r, accs, base=base):
                rr = base + r * 5
                a0, a1, a2, a3 = accs
                for u in range(5):
                    a0 = a0 + buf[rr + u, pl.ds(0, 16)]
                    a1 = a1 + buf[rr + u, pl.ds(16, 16)]
                    a2 = a2 + buf[rr + u, pl.ds(32, 16)]
                    a3 = a3 + buf[rr + u, pl.ds(48, 16)]
                return (a0, a1, a2, a3)

            z = jnp.zeros((16,), jnp.float32)
            accs = lax.fori_loop(0, _L // 5, body, (z, z, z, z))
            for j in range(4):
                acc[orow, pl.ds(j * 16, 16)] = accs[j]

    fire(0, b0, sem0)

    def body2(i, carry):
        c0 = 2 * i
        fire(c0 + 1, b1, sem1)
        drain(c0, b0, sem0)
        reduce_chunk(c0, b0)

        @pl.when(i < _CPW // 2 - 1)
        def _():
            fire(c0 + 2, b0, sem0)

        drain(c0 + 1, b1, sem1)
        reduce_chunk(c0 + 1, b1)
        return carry

    lax.fori_loop(0, _CPW // 2, body2, 0)
    pltpu.sync_copy(acc, out.at[pl.ds(wid * _RPW, _RPW)])


def _pool_word_table(x3, emb, out, row0, idxs, b0, b1, acc, zidx, r0buf,
                     sem0, sem1):
    wid = lax.axis_index("s") * _NC + lax.axis_index("c")

    @pl.when(wid == 0)
    def _():
        zidx[pl.ds(0, 16)] = jnp.zeros((16,), jnp.int32)
        pltpu.async_copy(emb.at[zidx.at[pl.ds(0, 8)]], r0buf, sem0)
        pltpu.make_async_copy(emb.at[zidx.at[pl.ds(0, 8)]], r0buf,
                              sem0).wait()
        pltpu.sync_copy(r0buf, row0)

    _pool_one_table(x3, emb, out, idxs, b0, b1, acc, sem0, sem1)


_SC_SCRATCH = [
    pltpu.VMEM((_CPW, _CI), jnp.int32),
    pltpu.VMEM((_CI, _D), jnp.float32),
    pltpu.VMEM((_CI, _D), jnp.float32),
    pltpu.VMEM((_RPW, _D), jnp.float32),
]

_sc_pool1 = functools.partial(
    pl.kernel,
    mesh=plsc.VectorSubcoreMesh(core_axis_name="c", subcore_axis_name="s"),
    out_type=[jax.ShapeDtypeStruct((_B, _D), jnp.float32),
              jax.ShapeDtypeStruct((8, _D), jnp.float32)],
    scratch_types=_SC_SCRATCH + [
        pltpu.VMEM((16,), jnp.int32),
        pltpu.VMEM((8, _D), jnp.float32),
        pltpu.SemaphoreType.DMA,
        pltpu.SemaphoreType.DMA,
    ],
    compiler_params=pltpu.CompilerParams(use_tc_tiling_on_sc=False),
)(_pool_word_table)

_sc_pool = functools.partial(
    pl.kernel,
    mesh=plsc.VectorSubcoreMesh(core_axis_name="c", subcore_axis_name="s"),
    out_type=jax.ShapeDtypeStruct((_B, _D), jnp.float32),
    scratch_types=_SC_SCRATCH + [
        pltpu.SemaphoreType.DMA,
        pltpu.SemaphoreType.DMA,
    ],
    compiler_params=pltpu.CompilerParams(use_tc_tiling_on_sc=False),
)(_pool_one_table)


def _tc_body(p1_ref, p2_ref, p3_ref, r0_ref, x_ref, fc1w_ref, fc1b_ref,
             gamma_ref, beta_ref, fc2w_ref, fc2b_ref, out_ref):
    x = x_ref[...]
    cnt0 = jnp.sum((x == 0).astype(jnp.float32), axis=1,
                   keepdims=True)
    inv_l = 1.0 / _L
    p1 = (p1_ref[...] - cnt0 * r0_ref[0:1, :]) * inv_l
    p2 = p2_ref[...] * inv_l
    p3 = p3_ref[...] * inv_l
    w = fc1w_ref[...]
    z = (lax.dot_general(p1, w[:, 0:_D], (((1,), (1,)), ((), ())),
                         preferred_element_type=jnp.float32)
         + lax.dot_general(p2, w[:, _D:2 * _D], (((1,), (1,)), ((), ())),
                           preferred_element_type=jnp.float32)
         + lax.dot_general(p3, w[:, 2 * _D:3 * _D], (((1,), (1,)), ((), ())),
                           preferred_element_type=jnp.float32))
    z = jnp.maximum(z + fc1b_ref[...], 0.0)
    m = jnp.mean(z, axis=0, keepdims=True)
    v = jnp.mean((z - m) * (z - m), axis=0, keepdims=True)
    zn = (z - m) * lax.rsqrt(v + 1e-5) * gamma_ref[...] + beta_ref[...]
    out_ref[...] = lax.dot_general(
        zn, fc2w_ref[...], (((1,), (1,)), ((), ())),
        preferred_element_type=jnp.float32) + fc2b_ref[...]


_tc_mlp = pl.pallas_call(
    _tc_body,
    grid=(1,),
    in_specs=[
        pl.BlockSpec((_B, _D), lambda i: (0, 0)),
        pl.BlockSpec((_B, _D), lambda i: (0, 0)),
        pl.BlockSpec((_B, _D), lambda i: (0, 0)),
        pl.BlockSpec((8, _D), lambda i: (0, 0)),
        pl.BlockSpec((_B, _L), lambda i: (0, 0)),
        pl.BlockSpec((_H, 3 * _D), lambda i: (0, 0)),
        pl.BlockSpec((1, _H), lambda i: (0, 0)),
        pl.BlockSpec((1, _H), lambda i: (0, 0)),
        pl.BlockSpec((1, _H), lambda i: (0, 0)),
        pl.BlockSpec((2, _H), lambda i: (0, 0)),
        pl.BlockSpec((1, 2), lambda i: (0, 0)),
    ],
    out_specs=pl.BlockSpec((_B, 2), lambda i: (0, 0)),
    out_shape=jax.ShapeDtypeStruct((_B, 2), jnp.float32),
)


def kernel(x, s, emb1, emb2, emb3, fc1_w, fc1_b, gamma, beta, fc2_w, fc2_b):
    x = x.astype(jnp.int32)
    xg = jnp.maximum(x.reshape(_NW, _CPW, _CI), 0)
    p1, r0 = _sc_pool1(xg, emb1)
    p2 = _sc_pool(xg, emb2)
    p3 = _sc_pool(xg, emb3)
    return _tc_mlp(p1, p2, p3, r0, x, fc1_w, fc1_b.reshape(1, _H),
                   gamma.reshape(1, _H), beta.reshape(1, _H), fc2_w,
                   fc2_b.reshape(1, 2))
